# baseline (speedup 1.0000x reference)
import jax
import jax.numpy as jnp
from jax import lax
from jax.experimental import pallas as pl
from jax.experimental.pallas import tpu as pltpu
from jax.experimental.pallas import tpu_sc as plsc

_N = 10000
_E = 320000
_F = 16
_B = 125
_NC = 2
_NS = 16
_NW = _NC * _NS
_NB = _E // _B
_BW = _NB // _NW
_DPT = 640
_ND = _DPT * _NS
_REM = _N - (_NS - 1) * _DPT

_mesh = plsc.VectorSubcoreMesh(
    core_axis_name="c", subcore_axis_name="s", num_cores=_NC, num_subcores=_NS
)
_sc_params = pltpu.CompilerParams(use_tc_tiling_on_sc=False)


def _zero_fill(ref, nrows):
    def _f(i, _):
        ref[i] = jnp.zeros((16,), jnp.float32)
        return 0

    lax.fori_loop(0, nrows, _f, 0, unroll=8)


def _stage_real_rows(src_hbm, dst, s, local=False):
    def _dst(size):
        return dst.at[pl.ds(0, size)] if local else dst.at[
            pl.ds(s * _DPT, size)]

    @pl.when(s < _NS - 1)
    def _():
        pltpu.sync_copy(src_hbm.at[pl.ds(s * _DPT, _DPT)], _dst(_DPT))

    @pl.when(s == _NS - 1)
    def _():
        pltpu.sync_copy(
            src_hbm.at[pl.ds((_NS - 1) * _DPT, _REM)], _dst(_REM)
        )


def _edge_loop(g_src, acc_sp, src_v, dst_v, rows_v, gsem, ssem):
    for j in range(3):
        pltpu.async_copy(g_src.at[src_v.at[j]], rows_v.at[j], gsem)

    def _quad(o, _):
        for b in range(4):
            j = 4 * o + b
            jm1 = jnp.maximum(j - 1, 0)

            @pl.when(j >= 1)
            def _():
                pltpu.make_async_copy(
                    rows_v.at[(b + 3) % 4], acc_sp.at[dst_v.at[jm1]], ssem
                ).wait()

            jg = jnp.minimum(j + 3, _BW - 1)

            @pl.when(j + 3 < _BW)
            def _():
                pltpu.async_copy(
                    g_src.at[src_v.at[jg]], rows_v.at[(b + 3) % 4], gsem
                )

            pltpu.make_async_copy(
                g_src.at[src_v.at[j]], rows_v.at[b], gsem
            ).wait()
            pltpu.async_copy(rows_v.at[b], acc_sp.at[dst_v.at[j]], ssem,
                             add=True)
        return 0

    lax.fori_loop(0, _BW // 4, _quad, 0)
    pltpu.make_async_copy(
        rows_v.at[3], acc_sp.at[dst_v.at[_BW - 1]], ssem
    ).wait()


def _deg_body(dst_hbm, deg_hbm, dst_v, ones_v, z_v, deg_sp, sem):
    c = lax.axis_index("c")
    s = lax.axis_index("s")
    w = s * _NC + c

    def _fill_zero(i, _):
        z_v[pl.ds(i * 16, 16)] = jnp.zeros((16,), jnp.float32)
        return 0

    lax.fori_loop(0, _DPT // 16, _fill_zero, 0, unroll=8)

    def _fill_one(i, _):
        ones_v[pl.ds(i * 16, 16)] = jnp.ones((16,), jnp.float32)
        return 0

    lax.fori_loop(0, 128 // 16, _fill_one, 0, unroll=8)

    pltpu.sync_copy(z_v, deg_sp.at[pl.ds(s * _DPT, _DPT)])
    pltpu.sync_copy(dst_hbm.at[pl.ds(w * _BW, _BW)], dst_v)
    plsc.subcore_barrier()

    def _batch8(o, _):
        descs = [
            pltpu.async_copy(
                ones_v.at[pl.ds(0, _B)], deg_sp.at[dst_v.at[o * 8 + k]],
                sem, add=True,
            )
            for k in range(8)
        ]
        for d in descs:
            d.wait()
        return 0

    lax.fori_loop(0, _BW // 8, _batch8, 0)
    plsc.subcore_barrier()
    pltpu.sync_copy(
        deg_sp.at[pl.ds(s * _DPT, _DPT)],
        deg_hbm.at[pl.ds(c * _ND + s * _DPT, _DPT)],
    )


_deg_call = pl.kernel(
    _deg_body,
    out_type=jax.ShapeDtypeStruct((_NC * _ND,), jnp.float32),
    mesh=_mesh,
    scratch_types=[
        pltpu.VMEM((_BW, _B), jnp.int32),
        pltpu.VMEM((128,), jnp.float32),
        pltpu.VMEM((_DPT,), jnp.float32),
        pltpu.VMEM_SHARED((_ND,), jnp.float32),
        pltpu.SemaphoreType.DMA,
    ],
    compiler_params=_sc_params,
)


def _agg1_body(g_hbm, src_hbm, dst_hbm, out_hbm, src_v, dst_v, rows_v, zr_v,
               acc_sp, gsem, ssem):
    c = lax.axis_index("c")
    s = lax.axis_index("s")
    w = s * _NC + c

    _zero_fill(zr_v, _DPT)
    pltpu.sync_copy(zr_v, acc_sp.at[pl.ds(s * _DPT, _DPT)])
    pltpu.sync_copy(src_hbm.at[pl.ds(w * _BW, _BW)], src_v)
    pltpu.sync_copy(dst_hbm.at[pl.ds(w * _BW, _BW)], dst_v)
    plsc.subcore_barrier()
    _edge_loop(g_hbm, acc_sp, src_v, dst_v, rows_v, gsem, ssem)
    plsc.subcore_barrier()
    pltpu.sync_copy(
        acc_sp.at[pl.ds(s * _DPT, _DPT)], out_hbm.at[c, pl.ds(s * _DPT, _DPT)]
    )


_agg1_call = pl.kernel(
    _agg1_body,
    out_type=jax.ShapeDtypeStruct((_NC, _ND, _F), jnp.float32),
    mesh=_mesh,
    scratch_types=[
        pltpu.VMEM((_BW, _B), jnp.int32),
        pltpu.VMEM((_BW, _B), jnp.int32),
        pltpu.VMEM((4, _B, _F), jnp.float32),
        pltpu.VMEM((_DPT, _F), jnp.float32),
        pltpu.VMEM_SHARED((_ND, _F), jnp.float32),
        pltpu.SemaphoreType.DMA,
        pltpu.SemaphoreType.DMA,
    ],
    compiler_params=_sc_params,
)


def _agg2_body(s1_hbm, g1_hbm, disb_hbm, b1_hbm, src_hbm, dst_hbm,
               out_hbm, g2_hbm,
               src_v, dst_v, rows_v, st0_v, st1_v, g1_v, db_v, g2_v, b1_v,
               acc_sp, gsem, ssem):
    c = lax.axis_index("c")
    s = lax.axis_index("s")
    w = s * _NC + c

    _zero_fill(g2_v, _DPT)
    pltpu.sync_copy(g2_v, acc_sp.at[pl.ds(s * _DPT, _DPT)])
    pltpu.sync_copy(src_hbm.at[pl.ds(w * _BW, _BW)], src_v)
    pltpu.sync_copy(dst_hbm.at[pl.ds(w * _BW, _BW)], dst_v)
    pltpu.sync_copy(s1_hbm.at[0, pl.ds(s * _DPT, _DPT)], st0_v)
    pltpu.sync_copy(s1_hbm.at[1, pl.ds(s * _DPT, _DPT)], st1_v)
    _stage_real_rows(g1_hbm, g1_v, s, local=True)
    _stage_real_rows(disb_hbm, db_v, s, local=True)
    pltpu.sync_copy(b1_hbm, b1_v)

    b1vec = b1_v[...]

    def _g2row(r, _):
        d = db_v[r]
        g2_v[r] = (d * (st0_v[r] + st1_v[r] + g1_v[r]) + b1vec) * d
        return 0

    lax.fori_loop(0, _DPT, _g2row, 0, unroll=8)
    pltpu.sync_copy(g2_v, g2_hbm.at[pl.ds(s * _DPT, _DPT)])
    plsc.subcore_barrier()
    _edge_loop(g2_hbm, acc_sp, src_v, dst_v, rows_v, gsem, ssem)
    plsc.subcore_barrier()
    pltpu.sync_copy(
        acc_sp.at[pl.ds(s * _DPT, _DPT)], out_hbm.at[c, pl.ds(s * _DPT, _DPT)]
    )


_agg2_call = pl.kernel(
    _agg2_body,
    out_type=(
        jax.ShapeDtypeStruct((_NC, _ND, _F), jnp.float32),
        jax.ShapeDtypeStruct((_ND, _F), jnp.float32),
    ),
    mesh=_mesh,
    scratch_types=[
        pltpu.VMEM((_BW, _B), jnp.int32),
        pltpu.VMEM((_BW, _B), jnp.int32),
        pltpu.VMEM((4, _B, _F), jnp.float32),
        pltpu.VMEM((_DPT, _F), jnp.float32),
        pltpu.VMEM((_DPT, _F), jnp.float32),
        pltpu.VMEM((_DPT, _F), jnp.float32),
        pltpu.VMEM((_DPT, _F), jnp.float32),
        pltpu.VMEM((_DPT, _F), jnp.float32),
        pltpu.VMEM((_F,), jnp.float32),
        pltpu.VMEM_SHARED((_ND, _F), jnp.float32),
        pltpu.SemaphoreType.DMA,
        pltpu.SemaphoreType.DMA,
    ],
    compiler_params=_sc_params,
)


def _dis_from(deg_ref):
    d = deg_ref[...]
    deg = d[:_N] + d[_ND : _ND + _N] + 1.0
    return lax.rsqrt(deg)[:, None]


def _tc1_body(x_ref, w_ref, deg_ref, g1_ref, disb_ref):
    dis = _dis_from(deg_ref)
    h = jnp.dot(x_ref[...], w_ref[...], preferred_element_type=jnp.float32)
    g1_ref[...] = h * dis
    disb_ref[...] = jnp.broadcast_to(dis, (_N, _F))


_tc1 = pl.pallas_call(
    _tc1_body,
    out_shape=(
        jax.ShapeDtypeStruct((_N, _F), jnp.float32),
        jax.ShapeDtypeStruct((_N, _F), jnp.float32),
    ),
)


def _tc3_body(s2_ref, g2_ref, deg_ref, w2_ref, b2_ref, out_ref):
    dis = _dis_from(deg_ref)
    z = dis * (s2_ref[0, :_N] + s2_ref[1, :_N] + g2_ref[:_N])
    out_ref[...] = (
        jnp.dot(z, w2_ref[...], preferred_element_type=jnp.float32)
        + b2_ref[...][None, :]
    )


_tc3 = pl.pallas_call(
    _tc3_body,
    out_shape=jax.ShapeDtypeStruct((_N, 40), jnp.float32),
)


def kernel(x, edge_index, W1, b1, W2, b2):
    src = edge_index[0].reshape(_NB, _B)
    dst = edge_index[1].reshape(_NB, _B)
    deg2 = _deg_call(dst)
    g1, disb = _tc1(x, W1, deg2)
    s1 = _agg1_call(g1, src, dst)
    s2, g2 = _agg2_call(s1, g1, disb, b1, src, dst)
    return _tc3(s2, g2, deg2, W2, b2)


# speedup vs baseline: 1.1581x; 1.1581x over previous
import jax
import jax.numpy as jnp
from jax import lax
from jax.experimental import pallas as pl
from jax.experimental.pallas import tpu as pltpu
from jax.experimental.pallas import tpu_sc as plsc

_N = 10000
_E = 320000
_F = 16
_B = 125
_NC = 2
_NS = 16
_NW = _NC * _NS
_NB = _E // _B
_BW = _NB // _NW
_DPT = 640
_ND = _DPT * _NS
_REM = _N - (_NS - 1) * _DPT

_mesh = plsc.VectorSubcoreMesh(
    core_axis_name="c", subcore_axis_name="s", num_cores=_NC, num_subcores=_NS
)
_sc_params = pltpu.CompilerParams(
    use_tc_tiling_on_sc=False, skip_device_barrier=True
)


def _zero_fill(ref, nrows):
    def _f(i, _):
        ref[i] = jnp.zeros((16,), jnp.float32)
        return 0

    lax.fori_loop(0, nrows, _f, 0, unroll=8)


def _stage_real_rows(src_hbm, dst, s, local=False):
    def _dst(size):
        return dst.at[pl.ds(0, size)] if local else dst.at[
            pl.ds(s * _DPT, size)]

    @pl.when(s < _NS - 1)
    def _():
        pltpu.sync_copy(src_hbm.at[pl.ds(s * _DPT, _DPT)], _dst(_DPT))

    @pl.when(s == _NS - 1)
    def _():
        pltpu.sync_copy(
            src_hbm.at[pl.ds((_NS - 1) * _DPT, _REM)], _dst(_REM)
        )


def _edge_loop(g_sp, acc_sp, src_v, dst_v, rows_v, gsem, ssem):
    pltpu.async_copy(g_sp.at[src_v.at[0]], rows_v.at[0], gsem)

    def _pair(o, _):
        j0 = 2 * o
        pltpu.make_async_copy(g_sp.at[src_v.at[j0]], rows_v.at[0], gsem).wait()
        pltpu.async_copy(g_sp.at[src_v.at[j0 + 1]], rows_v.at[1], gsem)
        pltpu.async_copy(rows_v.at[0], acc_sp.at[dst_v.at[j0]], ssem,
                         add=True).wait()
        pltpu.make_async_copy(
            g_sp.at[src_v.at[j0 + 1]], rows_v.at[1], gsem
        ).wait()
        pltpu.async_copy(g_sp.at[src_v.at[j0 + 2]], rows_v.at[0], gsem)
        pltpu.async_copy(rows_v.at[1], acc_sp.at[dst_v.at[j0 + 1]], ssem,
                         add=True).wait()
        return 0

    lax.fori_loop(0, _BW // 2 - 1, _pair, 0)
    pltpu.make_async_copy(
        g_sp.at[src_v.at[_BW - 2]], rows_v.at[0], gsem
    ).wait()
    pltpu.async_copy(g_sp.at[src_v.at[_BW - 1]], rows_v.at[1], gsem)
    pltpu.async_copy(rows_v.at[0], acc_sp.at[dst_v.at[_BW - 2]], ssem,
                     add=True).wait()
    pltpu.make_async_copy(
        g_sp.at[src_v.at[_BW - 1]], rows_v.at[1], gsem
    ).wait()
    pltpu.async_copy(rows_v.at[1], acc_sp.at[dst_v.at[_BW - 1]], ssem,
                     add=True).wait()


def _deg_body(dst_hbm, deg_hbm, dst_v, ones_v, z_v, deg_sp, sem):
    c = lax.axis_index("c")
    s = lax.axis_index("s")
    w = s * _NC + c

    def _fill_zero(i, _):
        z_v[pl.ds(i * 16, 16)] = jnp.zeros((16,), jnp.float32)
        return 0

    lax.fori_loop(0, _DPT // 16, _fill_zero, 0, unroll=8)

    def _fill_one(i, _):
        ones_v[pl.ds(i * 16, 16)] = jnp.ones((16,), jnp.float32)
        return 0

    lax.fori_loop(0, 128 // 16, _fill_one, 0, unroll=8)

    pltpu.sync_copy(z_v, deg_sp.at[pl.ds(s * _DPT, _DPT)])
    pltpu.sync_copy(dst_hbm.at[pl.ds(w * _BW, _BW)], dst_v)
    plsc.subcore_barrier()

    def _batch8(o, _):
        descs = [
            pltpu.async_copy(
                ones_v.at[pl.ds(0, _B)], deg_sp.at[dst_v.at[o * 8 + k]],
                sem, add=True,
            )
            for k in range(8)
        ]
        for d in descs:
            d.wait()
        return 0

    lax.fori_loop(0, _BW // 8, _batch8, 0)
    plsc.subcore_barrier()
    pltpu.sync_copy(
        deg_sp.at[pl.ds(s * _DPT, _DPT)],
        deg_hbm.at[pl.ds(c * _ND + s * _DPT, _DPT)],
    )


_deg_call = pl.kernel(
    _deg_body,
    out_type=jax.ShapeDtypeStruct((_NC * _ND,), jnp.float32),
    mesh=_mesh,
    scratch_types=[
        pltpu.VMEM((_BW, _B), jnp.int32),
        pltpu.VMEM((128,), jnp.float32),
        pltpu.VMEM((_DPT,), jnp.float32),
        pltpu.VMEM_SHARED((_ND,), jnp.float32),
        pltpu.SemaphoreType.DMA,
    ],
    compiler_params=_sc_params,
)


def _agg1_body(g_hbm, src_hbm, dst_hbm, out_hbm, src_v, dst_v, rows_v, zr_v,
               g_sp, acc_sp, gsem, ssem):
    c = lax.axis_index("c")
    s = lax.axis_index("s")
    w = s * _NC + c

    _zero_fill(zr_v, _DPT)
    pltpu.sync_copy(zr_v, acc_sp.at[pl.ds(s * _DPT, _DPT)])
    pltpu.sync_copy(src_hbm.at[pl.ds(w * _BW, _BW)], src_v)
    pltpu.sync_copy(dst_hbm.at[pl.ds(w * _BW, _BW)], dst_v)
    _stage_real_rows(g_hbm, g_sp, s)
    plsc.subcore_barrier()
    _edge_loop(g_sp, acc_sp, src_v, dst_v, rows_v, gsem, ssem)
    plsc.subcore_barrier()
    pltpu.sync_copy(
        acc_sp.at[pl.ds(s * _DPT, _DPT)], out_hbm.at[c, pl.ds(s * _DPT, _DPT)]
    )


_agg1_call = pl.kernel(
    _agg1_body,
    out_type=jax.ShapeDtypeStruct((_NC, _ND, _F), jnp.float32),
    mesh=_mesh,
    scratch_types=[
        pltpu.VMEM((_BW, _B), jnp.int32),
        pltpu.VMEM((_BW, _B), jnp.int32),
        pltpu.VMEM((2, _B, _F), jnp.float32),
        pltpu.VMEM((_DPT, _F), jnp.float32),
        pltpu.VMEM_SHARED((_ND, _F), jnp.float32),
        pltpu.VMEM_SHARED((_ND, _F), jnp.float32),
        pltpu.SemaphoreType.DMA,
        pltpu.SemaphoreType.DMA,
    ],
    compiler_params=_sc_params,
)


def _agg2_body(s1_hbm, g1_hbm, disb_hbm, b1_hbm, src_hbm, dst_hbm,
               out_hbm, g2_hbm,
               src_v, dst_v, rows_v, st0_v, st1_v, g1_v, db_v, g2_v, b1_v,
               g_sp, acc_sp, gsem, ssem):
    c = lax.axis_index("c")
    s = lax.axis_index("s")
    w = s * _NC + c

    _zero_fill(g2_v, _DPT)
    pltpu.sync_copy(g2_v, acc_sp.at[pl.ds(s * _DPT, _DPT)])
    pltpu.sync_copy(src_hbm.at[pl.ds(w * _BW, _BW)], src_v)
    pltpu.sync_copy(dst_hbm.at[pl.ds(w * _BW, _BW)], dst_v)
    pltpu.sync_copy(s1_hbm.at[0, pl.ds(s * _DPT, _DPT)], st0_v)
    pltpu.sync_copy(s1_hbm.at[1, pl.ds(s * _DPT, _DPT)], st1_v)
    _stage_real_rows(g1_hbm, g1_v, s, local=True)
    _stage_real_rows(disb_hbm, db_v, s, local=True)
    pltpu.sync_copy(b1_hbm, b1_v)

    b1vec = b1_v[...]

    def _g2row(r, _):
        d = db_v[r]
        g2_v[r] = (d * (st0_v[r] + st1_v[r] + g1_v[r]) + b1vec) * d
        return 0

    lax.fori_loop(0, _DPT, _g2row, 0, unroll=8)
    pltpu.sync_copy(g2_v, g_sp.at[pl.ds(s * _DPT, _DPT)])
    pltpu.sync_copy(g2_v, g2_hbm.at[pl.ds(s * _DPT, _DPT)])
    plsc.subcore_barrier()
    _edge_loop(g_sp, acc_sp, src_v, dst_v, rows_v, gsem, ssem)
    plsc.subcore_barrier()
    pltpu.sync_copy(
        acc_sp.at[pl.ds(s * _DPT, _DPT)], out_hbm.at[c, pl.ds(s * _DPT, _DPT)]
    )


_agg2_call = pl.kernel(
    _agg2_body,
    out_type=(
        jax.ShapeDtypeStruct((_NC, _ND, _F), jnp.float32),
        jax.ShapeDtypeStruct((_ND, _F), jnp.float32),
    ),
    mesh=_mesh,
    scratch_types=[
        pltpu.VMEM((_BW, _B), jnp.int32),
        pltpu.VMEM((_BW, _B), jnp.int32),
        pltpu.VMEM((2, _B, _F), jnp.float32),
        pltpu.VMEM((_DPT, _F), jnp.float32),
        pltpu.VMEM((_DPT, _F), jnp.float32),
        pltpu.VMEM((_DPT, _F), jnp.float32),
        pltpu.VMEM((_DPT, _F), jnp.float32),
        pltpu.VMEM((_DPT, _F), jnp.float32),
        pltpu.VMEM((_F,), jnp.float32),
        pltpu.VMEM_SHARED((_ND, _F), jnp.float32),
        pltpu.VMEM_SHARED((_ND, _F), jnp.float32),
        pltpu.SemaphoreType.DMA,
        pltpu.SemaphoreType.DMA,
    ],
    compiler_params=_sc_params,
)


def _dis_from(deg_ref):
    d = deg_ref[...]
    deg = d[:_N] + d[_ND : _ND + _N] + 1.0
    return lax.rsqrt(deg)[:, None]


def _tc1_body(x_ref, w_ref, deg_ref, g1_ref, disb_ref):
    dis = _dis_from(deg_ref)
    h = jnp.dot(x_ref[...], w_ref[...], preferred_element_type=jnp.float32)
    g1_ref[...] = h * dis
    disb_ref[...] = jnp.broadcast_to(dis, (_N, _F))


_tc1 = pl.pallas_call(
    _tc1_body,
    out_shape=(
        jax.ShapeDtypeStruct((_N, _F), jnp.float32),
        jax.ShapeDtypeStruct((_N, _F), jnp.float32),
    ),
)


def _tc3_body(s2_ref, g2_ref, deg_ref, w2_ref, b2_ref, out_ref):
    dis = _dis_from(deg_ref)
    z = dis * (s2_ref[0, :_N] + s2_ref[1, :_N] + g2_ref[:_N])
    out_ref[...] = (
        jnp.dot(z, w2_ref[...], preferred_element_type=jnp.float32)
        + b2_ref[...][None, :]
    )


_tc3 = pl.pallas_call(
    _tc3_body,
    out_shape=jax.ShapeDtypeStruct((_N, 40), jnp.float32),
)


def kernel(x, edge_index, W1, b1, W2, b2):
    src = edge_index[0].reshape(_NB, _B)
    dst = edge_index[1].reshape(_NB, _B)
    deg2 = _deg_call(dst)
    g1, disb = _tc1(x, W1, deg2)
    s1 = _agg1_call(g1, src, dst)
    s2, g2 = _agg2_call(s1, g1, disb, b1, src, dst)
    return _tc3(s2, g2, deg2, W2, b2)
